# baseline (speedup 1.0000x reference)
import functools

import jax
import jax.numpy as jnp
from jax import lax
from jax.experimental import pallas as pl
from jax.experimental.pallas import tpu as pltpu
from jax.experimental.pallas import tpu_sc as plsc

_POOL = 7
_PP = _POOL * _POOL
_NW = 32
_LANES = 16
_NPA = 26
_NPB = _PP - _NPA


def _precompute(boxes, image_shape):
    B, N = boxes.shape[0], boxes.shape[1]
    NB = B * N
    fb = boxes.reshape(NB, 4)
    y1, x1, y2, x2 = fb[:, 0], fb[:, 1], fb[:, 2], fb[:, 3]
    area = image_shape[0] * image_shape[1]
    rl = jnp.log2(jnp.sqrt((y2 - y1) * (x2 - x1)) / (224.0 / jnp.sqrt(area)))
    lvl = jnp.minimum(5, jnp.maximum(2, 4 + jnp.round(rl).astype(jnp.int32)))

    sizes = jnp.array([256, 128, 64, 32], jnp.int32)
    S = sizes[lvl - 2]
    Sf = S.astype(jnp.float32)
    g = jnp.arange(_POOL, dtype=jnp.float32) / (_POOL - 1)
    ys = y1[:, None] * (Sf - 1)[:, None] + g[None, :] * ((y2 - y1) * (Sf - 1))[:, None]
    xs = x1[:, None] * (Sf - 1)[:, None] + g[None, :] * ((x2 - x1) * (Sf - 1))[:, None]
    yg = jnp.clip(jnp.floor(ys).astype(jnp.int32), 0, (S - 2)[:, None])
    xg = jnp.clip(jnp.floor(xs).astype(jnp.int32), 0, (S - 2)[:, None])
    dy = ys - yg.astype(jnp.float32)
    dx = xs - xg.astype(jnp.float32)

    b = jnp.repeat(jnp.arange(B, dtype=jnp.int32), N)
    base = b * S * S
    i00 = (base[:, None, None] + yg[:, :, None] * S[:, None, None]
           + xg[:, None, :]).reshape(NB, _PP)
    Sb = S[:, None]
    corners = [i00, i00 + 1, i00 + Sb, i00 + Sb + 1]
    pad = jnp.zeros((NB, 3), jnp.int32)
    half_a = jnp.concatenate([c[:, :_NPA] for c in corners], axis=1)
    half_b = jnp.concatenate(
        sum([[c[:, _NPA:], pad] for c in corners], []), axis=1)
    gidx = jnp.stack([half_a, half_b], axis=1)

    wy0 = 1.0 - dy
    wx0 = 1.0 - dx
    w00 = (wy0[:, :, None] * wx0[:, None, :]).reshape(NB, _PP)
    w01 = (wy0[:, :, None] * dx[:, None, :]).reshape(NB, _PP)
    w10 = (dy[:, :, None] * wx0[:, None, :]).reshape(NB, _PP)
    w11 = (dy[:, :, None] * dx[:, None, :]).reshape(NB, _PP)
    wts = jnp.concatenate([w00, w01, w10, w11], axis=1)
    return gidx, wts, lvl


def _make_sc_kernel(NB, C):
    bpw = NB // _NW
    npairs = bpw // 2
    mesh = plsc.VectorSubcoreMesh(core_axis_name="c", subcore_axis_name="s")

    @functools.partial(
        pl.kernel,
        out_type=jax.ShapeDtypeStruct((NB * _PP * C,), jnp.float32),
        mesh=mesh,
        scratch_types=[
            pltpu.VMEM((bpw * 2, 104), jnp.int32),
            pltpu.VMEM((bpw * 4 * _PP,), jnp.float32),
            pltpu.VMEM((bpw,), jnp.int32),
            pltpu.VMEM((104, C // 2), jnp.int32),
            pltpu.VMEM((104, C // 2), jnp.int32),
            pltpu.VMEM((_PP * C,), jnp.float32),
            pltpu.VMEM((_PP * C,), jnp.float32),
            pltpu.SemaphoreType.DMA,
            pltpu.SemaphoreType.DMA,
            pltpu.SemaphoreType.DMA,
            pltpu.SemaphoreType.DMA,
        ],
        compiler_params=pltpu.CompilerParams(needs_layout_passes=False),
    )
    def sc_kernel(gidx_hbm, w_hbm, lvl_hbm, t2, t3, t4, t5, out_hbm,
                  idx_v, w_v, lvl_v, buf_a, buf_b, out_a, out_b,
                  sem_a, sem_b, sem_oa, sem_ob):
        wid = lax.axis_index("s") * 2 + lax.axis_index("c")
        pltpu.sync_copy(gidx_hbm.at[wid], idx_v)
        pltpu.sync_copy(w_hbm.at[wid], w_v)
        pltpu.sync_copy(lvl_hbm.at[wid], lvl_v)

        tables = ((2, t2), (3, t3), (4, t4), (5, t5))

        def lv_of(i):
            return jnp.max(plsc.load_gather(
                lvl_v, [jnp.full((_LANES,), i, jnp.int32)]))

        def issue_half(i, half, buf, sem):
            lv = lv_of(i)
            for l, tbl in tables:
                @pl.when(lv == l)
                def _():
                    pltpu.async_copy(tbl.at[idx_v.at[2 * i + half]], buf, sem)

        def drain_gather(buf, sem):
            pltpu.make_async_copy(t2.at[idx_v.at[0]], buf, sem).wait()

        def drain_out(outbuf, sem):
            pltpu.make_async_copy(
                outbuf, out_hbm.at[pl.ds(0, _PP * C)], sem).wait()

        lane2 = 2 * lax.iota(jnp.int32, _LANES)
        hi_mask = jnp.full((_LANES,), jnp.int32(-65536))

        def compute_half(i, buf, outbuf, pix0, npix, stride):
            @plsc.parallel_loop(0, npix, 1, unroll=2)
            def _pix(p):
                gp = pix0 + p
                wb = [plsc.load_gather(
                    w_v, [jnp.full((_LANES,), i * (4 * _PP) + c * _PP + gp,
                                   jnp.int32)])
                    for c in range(4)]
                for ch in range(C // (2 * _LANES)):
                    s = pl.ds(ch * _LANES, _LANES)
                    acc_lo = acc_hi = None
                    for c in range(4):
                        word = buf[c * stride + p, s]
                        lo = plsc.bitcast(lax.shift_left(word, 16),
                                          jnp.float32)
                        hi = plsc.bitcast(word & hi_mask, jnp.float32)
                        if acc_lo is None:
                            acc_lo = wb[c] * lo
                            acc_hi = wb[c] * hi
                        else:
                            acc_lo = acc_lo + wb[c] * lo
                            acc_hi = acc_hi + wb[c] * hi
                    base = jnp.full((_LANES,),
                                    gp * C + ch * (2 * _LANES),
                                    jnp.int32) + lane2
                    plsc.store_scatter(outbuf, [base], acc_lo)
                    plsc.store_scatter(outbuf, [base + 1], acc_hi)

        def do_box(i, j, outbuf, sem_out):
            drain_gather(buf_a, sem_a)
            @pl.when(j >= 1)
            def _():
                drain_out(outbuf, sem_out)
            compute_half(i, buf_a, outbuf, 0, _NPA, _NPA)
            @pl.when(i + 1 < bpw)
            def _():
                issue_half(i + 1, 0, buf_a, sem_a)
            drain_gather(buf_b, sem_b)
            compute_half(i, buf_b, outbuf, _NPA, _NPB, _NPA)
            @pl.when(i + 1 < bpw)
            def _():
                issue_half(i + 1, 1, buf_b, sem_b)
            base = (wid * bpw + i) * (_PP * C)
            pltpu.async_copy(outbuf, out_hbm.at[pl.ds(base, _PP * C)], sem_out)

        issue_half(0, 0, buf_a, sem_a)
        issue_half(0, 1, buf_b, sem_b)

        def pair_body(j, carry):
            do_box(2 * j, j, out_a, sem_oa)
            do_box(2 * j + 1, j, out_b, sem_ob)
            return carry

        lax.fori_loop(0, npairs, pair_body, 0)
        drain_out(out_a, sem_oa)
        drain_out(out_b, sem_ob)

    return sc_kernel


def kernel(boxes, image_shape, p2, p3, p4, p5):
    B, N = boxes.shape[0], boxes.shape[1]
    C = p2.shape[-1]
    NB = B * N
    bpw = NB // _NW

    gidx, wts, lvl = _precompute(boxes, image_shape)
    gidx = gidx.reshape(_NW, bpw * 2, 104)
    wts = wts.reshape(_NW, bpw * 4 * _PP)
    lvl = lvl.reshape(_NW, bpw)

    def pack(p):
        pb = p.astype(jnp.bfloat16).reshape(-1, C // 2, 2)
        return jax.lax.bitcast_convert_type(pb, jnp.int32)

    t2, t3, t4, t5 = pack(p2), pack(p3), pack(p4), pack(p5)

    out = _make_sc_kernel(NB, C)(gidx, wts, lvl, t2, t3, t4, t5)
    return out.reshape(B, N, _POOL, _POOL, C)


# speedup vs baseline: 2.4372x; 2.4372x over previous
import functools

import jax
import jax.numpy as jnp
from jax import lax
from jax.experimental import pallas as pl
from jax.experimental.pallas import tpu as pltpu
from jax.experimental.pallas import tpu_sc as plsc

_POOL = 7
_PP = _POOL * _POOL
_NW = 32
_LANES = 16
_NPA = 26
_NPB = _PP - _NPA


def _precompute(boxes, image_shape):
    B, N = boxes.shape[0], boxes.shape[1]
    NB = B * N
    fb = boxes.reshape(NB, 4)
    y1, x1, y2, x2 = fb[:, 0], fb[:, 1], fb[:, 2], fb[:, 3]
    area = image_shape[0] * image_shape[1]
    rl = jnp.log2(jnp.sqrt((y2 - y1) * (x2 - x1)) / (224.0 / jnp.sqrt(area)))
    lvl = jnp.minimum(5, jnp.maximum(2, 4 + jnp.round(rl).astype(jnp.int32)))

    sizes = jnp.array([256, 128, 64, 32], jnp.int32)
    S = sizes[lvl - 2]
    Sf = S.astype(jnp.float32)
    g = jnp.arange(_POOL, dtype=jnp.float32) / (_POOL - 1)
    ys = y1[:, None] * (Sf - 1)[:, None] + g[None, :] * ((y2 - y1) * (Sf - 1))[:, None]
    xs = x1[:, None] * (Sf - 1)[:, None] + g[None, :] * ((x2 - x1) * (Sf - 1))[:, None]
    yg = jnp.clip(jnp.floor(ys).astype(jnp.int32), 0, (S - 2)[:, None])
    xg = jnp.clip(jnp.floor(xs).astype(jnp.int32), 0, (S - 2)[:, None])
    dy = ys - yg.astype(jnp.float32)
    dx = xs - xg.astype(jnp.float32)

    b = jnp.repeat(jnp.arange(B, dtype=jnp.int32), N)
    base = b * S * S
    i00 = (base[:, None, None] + yg[:, :, None] * S[:, None, None]
           + xg[:, None, :]).reshape(NB, _PP)
    Sb = S[:, None]
    corners = [i00, i00 + 1, i00 + Sb, i00 + Sb + 1]
    pad = jnp.zeros((NB, 3), jnp.int32)
    half_a = jnp.concatenate([c[:, :_NPA] for c in corners], axis=1)
    half_b = jnp.concatenate(
        sum([[c[:, _NPA:], pad] for c in corners], []), axis=1)
    gidx = jnp.stack([half_a, half_b], axis=1)

    wy0 = 1.0 - dy
    wx0 = 1.0 - dx
    w00 = (wy0[:, :, None] * wx0[:, None, :]).reshape(NB, _PP)
    w01 = (wy0[:, :, None] * dx[:, None, :]).reshape(NB, _PP)
    w10 = (dy[:, :, None] * wx0[:, None, :]).reshape(NB, _PP)
    w11 = (dy[:, :, None] * dx[:, None, :]).reshape(NB, _PP)
    wts = jnp.concatenate([w00, w01, w10, w11], axis=1)
    return gidx, wts, lvl


def _make_sc_kernel(NB, C):
    bpw = NB // _NW
    npairs = bpw // 2
    mesh = plsc.VectorSubcoreMesh(core_axis_name="c", subcore_axis_name="s")

    @functools.partial(
        pl.kernel,
        out_type=jax.ShapeDtypeStruct((NB * _PP * C,), jnp.float32),
        mesh=mesh,
        scratch_types=[
            pltpu.VMEM((bpw * 2, 104), jnp.int32),
            pltpu.VMEM((bpw * 4 * _PP,), jnp.float32),
            pltpu.VMEM((bpw,), jnp.int32),
            pltpu.VMEM((104, C), jnp.float32),
            pltpu.VMEM((104, C), jnp.float32),
            pltpu.VMEM((_PP * C,), jnp.float32),
            pltpu.SemaphoreType.DMA,
            pltpu.SemaphoreType.DMA,
        ],
        compiler_params=pltpu.CompilerParams(needs_layout_passes=False),
    )
    def sc_kernel(gidx_hbm, w_hbm, lvl_hbm, t2, t3, t4, t5, out_hbm,
                  idx_v, w_v, lvl_v, buf_a, buf_b, out_a,
                  sem_a, sem_b):
        wid = lax.axis_index("s") * 2 + lax.axis_index("c")
        pltpu.sync_copy(gidx_hbm.at[wid], idx_v)
        pltpu.sync_copy(w_hbm.at[wid], w_v)
        pltpu.sync_copy(lvl_hbm.at[wid], lvl_v)

        tables = ((2, t2), (3, t3), (4, t4), (5, t5))

        def lv_of(i):
            return jnp.max(plsc.load_gather(
                lvl_v, [jnp.full((_LANES,), i, jnp.int32)]))

        def compute_half(i, buf, outbuf, pix0, npix, stride):
            @plsc.parallel_loop(0, npix, 1, unroll=2)
            def _pix(p):
                gp = pix0 + p
                wb = [plsc.load_gather(
                    w_v, [jnp.full((_LANES,), i * (4 * _PP) + c * _PP + gp,
                                   jnp.int32)])
                    for c in range(4)]
                for ch in range(C // _LANES):
                    s = pl.ds(ch * _LANES, _LANES)
                    acc = wb[0] * buf[p, s]
                    acc = acc + wb[1] * buf[stride + p, s]
                    acc = acc + wb[2] * buf[2 * stride + p, s]
                    acc = acc + wb[3] * buf[3 * stride + p, s]
                    outbuf[pl.ds(gp * C + ch * _LANES, _LANES)] = acc

        def box_body(i, carry):
            lv = lv_of(i)
            for l, tbl in tables:
                @pl.when(lv == l)
                def _():
                    c0 = pltpu.async_copy(tbl.at[idx_v.at[2 * i]],
                                          buf_a, sem_a)
                    c1 = pltpu.async_copy(tbl.at[idx_v.at[2 * i + 1]],
                                          buf_b, sem_b)
                    c0.wait()
                    c1.wait()
            compute_half(i, buf_a, out_a, 0, _NPA, _NPA)
            compute_half(i, buf_b, out_a, _NPA, _NPB, _NPA)
            base = (wid * bpw + i) * (_PP * C)
            pltpu.sync_copy(out_a, out_hbm.at[pl.ds(base, _PP * C)])
            return carry

        lax.fori_loop(0, bpw, box_body, 0)

    return sc_kernel


def kernel(boxes, image_shape, p2, p3, p4, p5):
    B, N = boxes.shape[0], boxes.shape[1]
    C = p2.shape[-1]
    NB = B * N
    bpw = NB // _NW

    gidx, wts, lvl = _precompute(boxes, image_shape)
    gidx = gidx.reshape(_NW, bpw * 2, 104)
    wts = wts.reshape(_NW, bpw * 4 * _PP)
    lvl = lvl.reshape(_NW, bpw)

    t2 = p2.reshape(-1, C)
    t3 = p3.reshape(-1, C)
    t4 = p4.reshape(-1, C)
    t5 = p5.reshape(-1, C)

    out = _make_sc_kernel(NB, C)(gidx, wts, lvl, t2, t3, t4, t5)
    return out.reshape(B, N, _POOL, _POOL, C)


# speedup vs baseline: 2.4585x; 1.0087x over previous
import functools

import jax
import jax.numpy as jnp
from jax import lax
from jax.experimental import pallas as pl
from jax.experimental.pallas import tpu as pltpu
from jax.experimental.pallas import tpu_sc as plsc

_POOL = 7
_PP = _POOL * _POOL
_NW = 32
_LANES = 16
_NPA = 26
_NPB = _PP - _NPA


def _precompute(boxes, image_shape):
    B, N = boxes.shape[0], boxes.shape[1]
    NB = B * N
    fb = boxes.reshape(NB, 4)
    y1, x1, y2, x2 = fb[:, 0], fb[:, 1], fb[:, 2], fb[:, 3]
    area = image_shape[0] * image_shape[1]
    rl = jnp.log2(jnp.sqrt((y2 - y1) * (x2 - x1)) / (224.0 / jnp.sqrt(area)))
    lvl = jnp.minimum(5, jnp.maximum(2, 4 + jnp.round(rl).astype(jnp.int32)))

    sizes = jnp.array([256, 128, 64, 32], jnp.int32)
    S = sizes[lvl - 2]
    Sf = S.astype(jnp.float32)
    g = jnp.arange(_POOL, dtype=jnp.float32) / (_POOL - 1)
    ys = y1[:, None] * (Sf - 1)[:, None] + g[None, :] * ((y2 - y1) * (Sf - 1))[:, None]
    xs = x1[:, None] * (Sf - 1)[:, None] + g[None, :] * ((x2 - x1) * (Sf - 1))[:, None]
    yg = jnp.clip(jnp.floor(ys).astype(jnp.int32), 0, (S - 2)[:, None])
    xg = jnp.clip(jnp.floor(xs).astype(jnp.int32), 0, (S - 2)[:, None])
    dy = ys - yg.astype(jnp.float32)
    dx = xs - xg.astype(jnp.float32)

    b = jnp.repeat(jnp.arange(B, dtype=jnp.int32), N)
    base = b * S * S
    i00 = (base[:, None, None] + yg[:, :, None] * S[:, None, None]
           + xg[:, None, :]).reshape(NB, _PP)
    Sb = S[:, None]
    corners = [i00, i00 + 1, i00 + Sb, i00 + Sb + 1]
    pad = jnp.zeros((NB, 3), jnp.int32)
    half_a = jnp.concatenate([c[:, :_NPA] for c in corners], axis=1)
    half_b = jnp.concatenate(
        sum([[c[:, _NPA:], pad] for c in corners], []), axis=1)
    gidx = jnp.stack([half_a, half_b], axis=1)

    wy0 = 1.0 - dy
    wx0 = 1.0 - dx
    w00 = (wy0[:, :, None] * wx0[:, None, :]).reshape(NB, _PP)
    w01 = (wy0[:, :, None] * dx[:, None, :]).reshape(NB, _PP)
    w10 = (dy[:, :, None] * wx0[:, None, :]).reshape(NB, _PP)
    w11 = (dy[:, :, None] * dx[:, None, :]).reshape(NB, _PP)
    wts = jnp.concatenate([w00, w01, w10, w11], axis=1)
    return gidx, wts, lvl


def _make_sc_kernel(NB, C):
    bpw = NB // _NW
    mesh = plsc.VectorSubcoreMesh(core_axis_name="c", subcore_axis_name="s")

    @functools.partial(
        pl.kernel,
        out_type=jax.ShapeDtypeStruct((NB * _PP * C,), jnp.float32),
        mesh=mesh,
        scratch_types=[
            pltpu.VMEM((bpw * 2, 104), jnp.int32),
            pltpu.VMEM((bpw * 4 * _PP,), jnp.float32),
            pltpu.VMEM((bpw,), jnp.int32),
            pltpu.VMEM((104, C), jnp.float32),
            pltpu.VMEM((104, C), jnp.float32),
            pltpu.VMEM((_PP * C,), jnp.float32),
            pltpu.SemaphoreType.DMA,
            pltpu.SemaphoreType.DMA,
        ],
        compiler_params=pltpu.CompilerParams(needs_layout_passes=False),
    )
    def sc_kernel(gidx_hbm, w_hbm, lvl_hbm, t2, t3, t4, t5, out_hbm,
                  idx_v, w_v, lvl_v, buf_a, buf_b, out_a,
                  sem_a, sem_b):
        wid = lax.axis_index("s") * 2 + lax.axis_index("c")
        pltpu.sync_copy(gidx_hbm.at[wid], idx_v)
        pltpu.sync_copy(w_hbm.at[wid], w_v)
        pltpu.sync_copy(lvl_hbm.at[wid], lvl_v)

        tables = ((2, t2), (3, t3), (4, t4), (5, t5))

        def lv_of(i):
            return jnp.max(plsc.load_gather(
                lvl_v, [jnp.full((_LANES,), i, jnp.int32)]))

        def compute_half(i, buf, outbuf, pix0, npix, stride):
            @plsc.parallel_loop(0, npix, 1, unroll=2)
            def _pix(p):
                gp = pix0 + p
                wb = [plsc.load_gather(
                    w_v, [jnp.full((_LANES,), i * (4 * _PP) + c * _PP + gp,
                                   jnp.int32)])
                    for c in range(4)]
                for ch in range(C // _LANES):
                    s = pl.ds(ch * _LANES, _LANES)
                    acc = wb[0] * buf[p, s]
                    acc = acc + wb[1] * buf[stride + p, s]
                    acc = acc + wb[2] * buf[2 * stride + p, s]
                    acc = acc + wb[3] * buf[3 * stride + p, s]
                    outbuf[pl.ds(gp * C + ch * _LANES, _LANES)] = acc

        def box_body(i, carry):
            lv = lv_of(i)
            for l, tbl in tables:
                @pl.when(lv == l)
                def _():
                    c0 = pltpu.async_copy(tbl.at[idx_v.at[2 * i]],
                                          buf_a, sem_a)
                    c1 = pltpu.async_copy(tbl.at[idx_v.at[2 * i + 1]],
                                          buf_b, sem_b)
                    c0.wait()
                    c1.wait()
            compute_half(i, buf_a, out_a, 0, _NPA, _NPA)
            compute_half(i, buf_b, out_a, _NPA, _NPB, _NPA)
            base = (wid * bpw + i) * (_PP * C)
            pltpu.sync_copy(out_a, out_hbm.at[pl.ds(base, _PP * C)])
            return carry

        lax.fori_loop(0, bpw, box_body, 0)

    return sc_kernel


def kernel(boxes, image_shape, p2, p3, p4, p5):
    B, N = boxes.shape[0], boxes.shape[1]
    C = p2.shape[-1]
    NB = B * N
    bpw = NB // _NW

    gidx, wts, lvl = _precompute(boxes, image_shape)
    gidx = gidx.reshape(_NW, bpw * 2, 104)
    wts = wts.reshape(_NW, bpw * 4 * _PP)
    lvl = lvl.reshape(_NW, bpw)

    t2 = p2.reshape(-1, C)
    t3 = p3.reshape(-1, C)
    t4 = p4.reshape(-1, C)
    t5 = p5.reshape(-1, C)

    out = _make_sc_kernel(NB, C)(gidx, wts, lvl, t2, t3, t4, t5)
    return out.reshape(B, N, _POOL, _POOL, C)
